# baseline (speedup 1.0000x reference)
import functools

import jax
import jax.numpy as jnp
from jax import lax
from jax.experimental import pallas as pl
from jax.experimental.pallas import tpu as pltpu
from jax.experimental.pallas import tpu_sc as plsc

INV_T = 2.0
G = 16
NGROUPS = 16384
CAP = 2048
K = 1024

_INTERPRET = False
_SC_STAGE = 99


def _stats_body(x_ref, m_ref, map_ref, gmax_ref, my_ref, denom_ref):
    y = (x_ref[...] + m_ref[...]) * INV_T
    H, W = y.shape[1], y.shape[2]
    g = jnp.max(y.reshape(1, H, W // G, G), axis=-1)
    gmax_ref[...] = g
    my = jnp.max(g)
    e = jnp.exp(y - my)
    s = jnp.sum(e)
    my_ref[...] = jnp.full((1, 1, 16), my, jnp.float32)
    denom_ref[...] = jnp.full((1, 1, 16), s, jnp.float32)
    map_ref[...] = e * (1.0 / s)


def _stage_a(sal, msk):
    B, H, W = sal.shape
    return pl.pallas_call(
        _stats_body,
        grid=(B,),
        in_specs=[
            pl.BlockSpec((1, H, W), lambda b: (b, 0, 0)),
            pl.BlockSpec((1, H, W), lambda b: (0, 0, 0)),
        ],
        out_specs=[
            pl.BlockSpec((1, H, W), lambda b: (b, 0, 0)),
            pl.BlockSpec((1, H, W // G), lambda b: (b, 0, 0)),
            pl.BlockSpec((1, 1, 16), lambda b: (b, 0, 0)),
            pl.BlockSpec((1, 1, 16), lambda b: (b, 0, 0)),
        ],
        out_shape=[
            jax.ShapeDtypeStruct((B, H, W), jnp.float32),
            jax.ShapeDtypeStruct((B, H, W // G), jnp.float32),
            jax.ShapeDtypeStruct((B, 1, 16), jnp.float32),
            jax.ShapeDtypeStruct((B, 1, 16), jnp.float32),
        ],
        compiler_params=pltpu.CompilerParams(dimension_semantics=("parallel",)),
        interpret=_INTERPRET,
    )(sal, msk)


def _bisect_body(gmax_ref, t_ref):
    g = gmax_ref[...]
    B = g.shape[0]
    lo = jnp.min(g, axis=(1, 2), keepdims=True)
    hi = jnp.max(g, axis=(1, 2), keepdims=True)

    def it(_, carry):
        lo, hi = carry
        mid = (lo + hi) * 0.5
        cnt = jnp.sum((g >= mid).astype(jnp.float32), axis=(1, 2), keepdims=True)
        pred = cnt >= K
        return jnp.where(pred, mid, lo), jnp.where(pred, hi, mid)

    lo2, _ = lax.fori_loop(0, 40, it, (lo, hi))
    t_ref[...] = jnp.broadcast_to(lo2.reshape(B, 1, 1), (B, 1, 16))


def _stage_b(gmax):
    B = gmax.shape[0]
    return pl.pallas_call(
        _bisect_body,
        out_shape=jax.ShapeDtypeStruct((B, 1, 16), jnp.float32),
        interpret=_INTERPRET,
    )(gmax)


def _stage_c_sc(sal, msk, gmax, t):
    B, H, W = sal.shape
    N = H * W
    info = plsc.get_sparse_core_info()
    NC, NS, L = info.num_cores, info.num_subcores, info.num_lanes
    NW = NC * NS
    RPW = B // NW

    gmax2 = gmax.reshape(B, NGROUPS)
    t16 = t.reshape(B, 16)
    sal_tab = sal.reshape(B * NGROUPS, G)
    msk_tab = msk.reshape(NGROUPS, G)
    NCHUNK = CAP // 128

    mesh = plsc.VectorSubcoreMesh(core_axis_name="c", subcore_axis_name="s")

    @functools.partial(
        pl.kernel, mesh=mesh,
        compiler_params=pltpu.CompilerParams(needs_layout_passes=False, use_tc_tiling_on_sc=False),
        out_type=[
            jax.ShapeDtypeStruct((B, CAP), jnp.float32),
            jax.ShapeDtypeStruct((B, CAP), jnp.int32),
        ],
        scratch_types=[
            pltpu.VMEM((NGROUPS,), jnp.float32),
            pltpu.VMEM((16,), jnp.float32),
            pltpu.VMEM((CAP,), jnp.int32),
            pltpu.VMEM((NCHUNK, 128), jnp.int32),
            pltpu.VMEM((NCHUNK, 128), jnp.int32),
            pltpu.VMEM((CAP, G), jnp.float32),
            pltpu.VMEM((CAP, G), jnp.float32),
            pltpu.VMEM((CAP,), jnp.float32),
            pltpu.VMEM((CAP,), jnp.int32),
            pltpu.SemaphoreType.DMA,
        ],
    )
    def sc_kern(gmax_h, t_h, sal_h, msk_h, cv_h, ci_h,
                gmax_v, t_v, idsl_v, idg_v, idl_v, grp_v, mgrp_v,
                cv_v, ci_v, sem):
        wid = lax.axis_index("s") * NC + lax.axis_index("c")
        iota16 = lax.iota(jnp.int32, 16)
        for r2 in range(RPW):
            r = wid * RPW + r2
            pltpu.sync_copy(gmax_h.at[r], gmax_v)
            pltpu.sync_copy(t_h.at[r], t_v)
            tvec = t_v[...]

            @plsc.parallel_loop(0, CAP // 16, unroll=8)
            def pre(i):
                idsl_v[pl.ds(i * 16, 16)] = jnp.zeros((16,), jnp.int32)
                cv_v[pl.ds(i * 16, 16)] = jnp.full((16,), -jnp.inf, jnp.float32)
                ci_v[pl.ds(i * 16, 16)] = jnp.full((16,), 2**30, jnp.int32)

            if _SC_STAGE < 1:
                continue
            @plsc.parallel_loop(0, NGROUPS // 16, unroll=8, carry=jnp.int32(0))
            def comp(i, off):
                gv = gmax_v[pl.ds(i * 16, 16)]
                m = gv >= tvec
                gid = iota16 + i * 16
                cs = plsc.cumsum(m.astype(jnp.int32))
                pos = jnp.minimum(off + cs - 1, CAP - 1)
                plsc.store_scatter(idsl_v, [pos], gid, mask=m)
                return off + jnp.max(cs)
            nflag = jnp.minimum(comp, CAP)

            if _SC_STAGE < 2:
                continue
            rbase = r * NGROUPS
            @plsc.parallel_loop(0, CAP // 16, unroll=8)
            def bld(i):
                v = idsl_v[pl.ds(i * 16, 16)]
                c = i // 8
                s8 = lax.rem(i, 8)
                idl_v[c, pl.ds(s8 * 16, 16)] = v
                idg_v[c, pl.ds(s8 * 16, 16)] = v + rbase

            if _SC_STAGE < 3:
                continue
            cps = []
            for c in range(NCHUNK):
                @pl.when(c * 128 < nflag)
                def _():
                    pltpu.async_copy(sal_h.at[idg_v.at[c]],
                                     grp_v.at[pl.ds(c * 128, 128)], sem)
                    pltpu.async_copy(msk_h.at[idl_v.at[c]],
                                     mgrp_v.at[pl.ds(c * 128, 128)], sem)
            for c in range(NCHUNK):
                @pl.when(c * 128 < nflag)
                def _():
                    pltpu.make_async_copy(sal_h.at[idg_v.at[c]],
                                          grp_v.at[pl.ds(c * 128, 128)], sem).wait()
                    pltpu.make_async_copy(msk_h.at[idl_v.at[c]],
                                          mgrp_v.at[pl.ds(c * 128, 128)], sem).wait()

            if _SC_STAGE < 4:
                continue
            ntrip = lax.div(nflag + 15, jnp.int32(16))
            @plsc.parallel_loop(0, ntrip, unroll=2, carry=jnp.int32(0))
            def filt(i, coff):
                iv = idsl_v[pl.ds(i * 16, 16)]
                for j in range(16):
                    slot = i * 16 + j
                    sv = jnp.broadcast_to(slot, (16,)).astype(jnp.int32)
                    gidb = jnp.take(iv, jnp.full((16,), j, jnp.int32))
                    y = (grp_v[slot] + mgrp_v[slot]) * INV_T
                    m = (y >= tvec) & (sv < nflag)
                    fidx = gidb * G + iota16
                    cs = plsc.cumsum(m.astype(jnp.int32))
                    pos = jnp.minimum(coff + cs - 1, CAP - 1)
                    plsc.store_scatter(cv_v, [pos], y, mask=m)
                    plsc.store_scatter(ci_v, [pos], fidx, mask=m)
                    coff = coff + jnp.max(cs)
                return coff
            _ = filt

            pltpu.sync_copy(cv_v, cv_h.at[r])
            pltpu.sync_copy(ci_v, ci_h.at[r])

    return sc_kern(gmax2, t16, sal_tab, msk_tab)


def _stage_c_stub(sal, msk, gmax, t):
    B, H, W = sal.shape
    N = H * W
    y = ((sal + msk).reshape(B, N)) * INV_T
    gm = gmax.reshape(B, NGROUPS)
    tt = t[:, 0, 0:1]
    flag = gm >= tt
    nflag = jnp.sum(flag.astype(jnp.int32), axis=1)
    order = jnp.argsort(~flag, axis=1, stable=True)
    ids = order[:, :CAP]
    slot = jnp.arange(CAP)[None, :]
    valid_g = slot < nflag[:, None]
    base = ids * G
    idxs = base[:, :, None] + jnp.arange(G)[None, None, :]
    iv = idxs.reshape(B, CAP * G)
    yv = jnp.take_along_axis(y, iv, axis=1)
    m = (yv >= tt) & jnp.repeat(valid_g, G, axis=1)
    ordc = jnp.argsort(~m, axis=1, stable=True)[:, :CAP]
    cv = jnp.where(jnp.take_along_axis(m, ordc, axis=1),
                   jnp.take_along_axis(yv, ordc, axis=1), -jnp.inf)
    ci = jnp.where(jnp.take_along_axis(m, ordc, axis=1),
                   jnp.take_along_axis(iv, ordc, axis=1), jnp.int32(2**30))
    return cv, ci.astype(jnp.int32)


def _sort_body(cv_ref, ci_ref, my_ref, denom_ref, sc_ref, si_ref):
    q = jnp.exp(cv_ref[...] - my_ref[...][:, :, 0:1]) / denom_ref[...][:, :, 0:1]
    idx = ci_ref[...]
    Bb = q.shape[0]

    sub = lax.broadcasted_iota(jnp.int32, (1, 16, 128), 1)
    lane = lax.broadcasted_iota(jnp.int32, (1, 16, 128), 2)
    n = sub * 128 + lane

    def cmpx(q, idx, d, k):
        if d < 128:
            pq = jnp.where((lane & d) == 0, jnp.roll(q, -d, axis=2),
                           jnp.roll(q, d, axis=2))
            pi = jnp.where((lane & d) == 0, jnp.roll(idx, -d, axis=2),
                           jnp.roll(idx, d, axis=2))
        else:
            ds = d // 128
            pq = jnp.where((sub & ds) == 0, jnp.roll(q, -ds, axis=1),
                           jnp.roll(q, ds, axis=1))
            pi = jnp.where((sub & ds) == 0, jnp.roll(idx, -ds, axis=1),
                           jnp.roll(idx, ds, axis=1))
        self_wins = (q > pq) | ((q == pq) & (idx < pi))
        wq = jnp.where(self_wins, q, pq)
        wi = jnp.where(self_wins, idx, pi)
        lq = jnp.where(self_wins, pq, q)
        li = jnp.where(self_wins, pi, idx)
        is_lo = (n & d) == 0
        desc = (n & (2 * k)) == 0
        takew = is_lo == desc
        return jnp.where(takew, wq, lq), jnp.where(takew, wi, li)

    k = 1
    while k < 2048:
        d = k
        while d >= 1:
            q, idx = cmpx(q, idx, d, k)
            d //= 2
        k *= 2

    sc_ref[...] = q[:, 0:8, :]
    si_ref[...] = idx[:, 0:8, :]


def _stage_d(cv, ci, my, denom):
    B = cv.shape[0]
    return pl.pallas_call(
        _sort_body,
        out_shape=[
            jax.ShapeDtypeStruct((B, 8, 128), jnp.float32),
            jax.ShapeDtypeStruct((B, 8, 128), jnp.int32),
        ],
        interpret=_INTERPRET,
    )(cv.reshape(B, 16, 128), ci.reshape(B, 16, 128), my, denom)


def kernel(saliency_map, K_arg, mask_logits):
    B, H, W = saliency_map.shape
    soft, gmax, my, denom = _stage_a(saliency_map, mask_logits)
    t = _stage_b(gmax)
    cv, ci = _stage_c_sc(saliency_map, mask_logits, gmax, t)
    scores, indices = _stage_d(cv, ci, my, denom)
    return (scores.reshape(B, K), indices.reshape(B, K), soft)


# speedup vs baseline: 1.0502x; 1.0502x over previous
import functools

import jax
import jax.numpy as jnp
from jax import lax
from jax.experimental import pallas as pl
from jax.experimental.pallas import tpu as pltpu
from jax.experimental.pallas import tpu_sc as plsc

INV_T = 2.0
G = 16
NGROUPS = 16384
CAP = 2048
K = 1024

_INTERPRET = False


def _stats_body(x_ref, m_ref, map_ref, gmax_ref, my_ref, denom_ref):
    y = (x_ref[...] + m_ref[...]) * INV_T
    H, W = y.shape[1], y.shape[2]
    gmax_ref[...] = jnp.max(y.reshape(1, H, W // G, G), axis=-1)
    my = jnp.max(y)
    e = jnp.exp(y - my)
    s = jnp.sum(e)
    my_ref[...] = jnp.full((1, 1, 16), my, jnp.float32)
    denom_ref[...] = jnp.full((1, 1, 16), s, jnp.float32)
    map_ref[...] = e * (1.0 / s)


def _stage_a(sal, msk):
    B, H, W = sal.shape
    return pl.pallas_call(
        _stats_body,
        grid=(B,),
        in_specs=[
            pl.BlockSpec((1, H, W), lambda b: (b, 0, 0)),
            pl.BlockSpec((1, H, W), lambda b: (0, 0, 0)),
        ],
        out_specs=[
            pl.BlockSpec((1, H, W), lambda b: (b, 0, 0)),
            pl.BlockSpec((1, H, W // G), lambda b: (b, 0, 0)),
            pl.BlockSpec((1, 1, 16), lambda b: (b, 0, 0)),
            pl.BlockSpec((1, 1, 16), lambda b: (b, 0, 0)),
        ],
        out_shape=[
            jax.ShapeDtypeStruct((B, H, W), jnp.float32),
            jax.ShapeDtypeStruct((B, H, W // G), jnp.float32),
            jax.ShapeDtypeStruct((B, 1, 16), jnp.float32),
            jax.ShapeDtypeStruct((B, 1, 16), jnp.float32),
        ],
        compiler_params=pltpu.CompilerParams(dimension_semantics=("parallel",)),
        interpret=_INTERPRET,
    )(sal, msk)


def _bisect_body(gmax_ref, t_ref):
    g = gmax_ref[...]
    B = g.shape[0]
    lo = jnp.min(g, axis=(1, 2), keepdims=True)
    hi = jnp.max(g, axis=(1, 2), keepdims=True)

    def it(_, carry):
        lo, hi = carry
        mid = (lo + hi) * 0.5
        cnt = jnp.sum((g >= mid).astype(jnp.float32), axis=(1, 2), keepdims=True)
        pred = cnt >= K
        return jnp.where(pred, mid, lo), jnp.where(pred, hi, mid)

    lo2, _ = lax.fori_loop(0, 40, it, (lo, hi))
    t_ref[...] = jnp.broadcast_to(lo2.reshape(B, 1, 1), (B, 1, 16))


def _stage_b(gmax):
    B = gmax.shape[0]
    return pl.pallas_call(
        _bisect_body,
        out_shape=jax.ShapeDtypeStruct((B, 1, 16), jnp.float32),
        interpret=_INTERPRET,
    )(gmax)


def _stage_c_sc(sal, msk, gmax, t):
    B, H, W = sal.shape
    N = H * W
    info = plsc.get_sparse_core_info()
    NC, NS, L = info.num_cores, info.num_subcores, info.num_lanes
    NW = NC * NS
    RPW = B // NW

    gmax2 = gmax.reshape(B, NGROUPS)
    t16 = t.reshape(B, 16)
    sal_tab = sal.reshape(B * NGROUPS, G)
    msk_tab = msk.reshape(NGROUPS, G)
    NCHUNK = CAP // 128

    mesh = plsc.VectorSubcoreMesh(core_axis_name="c", subcore_axis_name="s")

    @functools.partial(
        pl.kernel, mesh=mesh,
        compiler_params=pltpu.CompilerParams(needs_layout_passes=False, use_tc_tiling_on_sc=False),
        out_type=[
            jax.ShapeDtypeStruct((B, CAP), jnp.float32),
            jax.ShapeDtypeStruct((B, CAP), jnp.int32),
        ],
        scratch_types=[
            pltpu.VMEM((NGROUPS,), jnp.float32),
            pltpu.VMEM((16,), jnp.float32),
            pltpu.VMEM((CAP,), jnp.int32),
            pltpu.VMEM((NCHUNK, 128), jnp.int32),
            pltpu.VMEM((NCHUNK, 128), jnp.int32),
            pltpu.VMEM((CAP, G), jnp.float32),
            pltpu.VMEM((CAP, G), jnp.float32),
            pltpu.VMEM((CAP,), jnp.float32),
            pltpu.VMEM((CAP,), jnp.int32),
            pltpu.SemaphoreType.DMA,
        ],
    )
    def sc_kern(gmax_h, t_h, sal_h, msk_h, cv_h, ci_h,
                gmax_v, t_v, idsl_v, idg_v, idl_v, grp_v, mgrp_v,
                cv_v, ci_v, sem):
        wid = lax.axis_index("s") * NC + lax.axis_index("c")
        iota16 = lax.iota(jnp.int32, 16)
        for r2 in range(RPW):
            r = wid * RPW + r2
            pltpu.sync_copy(gmax_h.at[r], gmax_v)
            pltpu.sync_copy(t_h.at[r], t_v)
            tvec = t_v[...]

            @plsc.parallel_loop(0, CAP // 16, unroll=8)
            def pre(i):
                idsl_v[pl.ds(i * 16, 16)] = jnp.zeros((16,), jnp.int32)
                cv_v[pl.ds(i * 16, 16)] = jnp.full((16,), -jnp.inf, jnp.float32)
                ci_v[pl.ds(i * 16, 16)] = jnp.full((16,), 2**30, jnp.int32)

            @plsc.parallel_loop(0, NGROUPS // 16, unroll=8, carry=jnp.int32(0))
            def comp(i, off):
                gv = gmax_v[pl.ds(i * 16, 16)]
                m = gv >= tvec
                gid = iota16 + i * 16
                cs = plsc.cumsum(m.astype(jnp.int32))
                pos = jnp.minimum(off + cs - 1, CAP - 1)
                plsc.store_scatter(idsl_v, [pos], gid, mask=m)
                return off + jnp.max(cs)
            nflag = jnp.minimum(comp, CAP)

            rbase = r * NGROUPS
            @plsc.parallel_loop(0, CAP // 16, unroll=8)
            def bld(i):
                v = idsl_v[pl.ds(i * 16, 16)]
                c = i // 8
                s8 = lax.rem(i, 8)
                idl_v[c, pl.ds(s8 * 16, 16)] = v
                idg_v[c, pl.ds(s8 * 16, 16)] = v + rbase

            cps = []
            for c in range(NCHUNK):
                @pl.when(c * 128 < nflag)
                def _():
                    pltpu.async_copy(sal_h.at[idg_v.at[c]],
                                     grp_v.at[pl.ds(c * 128, 128)], sem)
                    pltpu.async_copy(msk_h.at[idl_v.at[c]],
                                     mgrp_v.at[pl.ds(c * 128, 128)], sem)
            for c in range(NCHUNK):
                @pl.when(c * 128 < nflag)
                def _():
                    pltpu.make_async_copy(sal_h.at[idg_v.at[c]],
                                          grp_v.at[pl.ds(c * 128, 128)], sem).wait()
                    pltpu.make_async_copy(msk_h.at[idl_v.at[c]],
                                          mgrp_v.at[pl.ds(c * 128, 128)], sem).wait()

            ntrip = lax.div(nflag + 15, jnp.int32(16))
            @plsc.parallel_loop(0, ntrip, unroll=2, carry=jnp.int32(0))
            def filt(i, coff):
                iv = idsl_v[pl.ds(i * 16, 16)]
                for j in range(16):
                    slot = i * 16 + j
                    sv = jnp.broadcast_to(slot, (16,)).astype(jnp.int32)
                    gidb = jnp.take(iv, jnp.full((16,), j, jnp.int32))
                    y = (grp_v[slot] + mgrp_v[slot]) * INV_T
                    m = (y >= tvec) & (sv < nflag)
                    fidx = gidb * G + iota16
                    cs = plsc.cumsum(m.astype(jnp.int32))
                    pos = jnp.minimum(coff + cs - 1, CAP - 1)
                    plsc.store_scatter(cv_v, [pos], y, mask=m)
                    plsc.store_scatter(ci_v, [pos], fidx, mask=m)
                    coff = coff + jnp.max(cs)
                return coff
            _ = filt

            pltpu.sync_copy(cv_v, cv_h.at[r])
            pltpu.sync_copy(ci_v, ci_h.at[r])

    return sc_kern(gmax2, t16, sal_tab, msk_tab)


def _sort_body(cv_ref, ci_ref, my_ref, denom_ref, sc_ref, si_ref):
    q = jnp.exp(cv_ref[...] - my_ref[...][:, :, 0:1]) / denom_ref[...][:, :, 0:1]
    idx = ci_ref[...]
    Bb = q.shape[0]

    sub = lax.broadcasted_iota(jnp.int32, (1, 16, 128), 1)
    lane = lax.broadcasted_iota(jnp.int32, (1, 16, 128), 2)
    n = sub * 128 + lane

    def cmpx(q, idx, d, k):
        if d < 128:
            pq = jnp.where((lane & d) == 0, jnp.roll(q, -d, axis=2),
                           jnp.roll(q, d, axis=2))
            pi = jnp.where((lane & d) == 0, jnp.roll(idx, -d, axis=2),
                           jnp.roll(idx, d, axis=2))
        else:
            ds = d // 128
            pq = jnp.where((sub & ds) == 0, jnp.roll(q, -ds, axis=1),
                           jnp.roll(q, ds, axis=1))
            pi = jnp.where((sub & ds) == 0, jnp.roll(idx, -ds, axis=1),
                           jnp.roll(idx, ds, axis=1))
        self_wins = (q > pq) | ((q == pq) & (idx < pi))
        wq = jnp.where(self_wins, q, pq)
        wi = jnp.where(self_wins, idx, pi)
        lq = jnp.where(self_wins, pq, q)
        li = jnp.where(self_wins, pi, idx)
        is_lo = (n & d) == 0
        desc = (n & (2 * k)) == 0
        takew = is_lo == desc
        return jnp.where(takew, wq, lq), jnp.where(takew, wi, li)

    k = 1
    while k < 2048:
        d = k
        while d >= 1:
            q, idx = cmpx(q, idx, d, k)
            d //= 2
        k *= 2

    sc_ref[...] = q[:, 0:8, :]
    si_ref[...] = idx[:, 0:8, :]


def _stage_d(cv, ci, my, denom):
    B = cv.shape[0]
    return pl.pallas_call(
        _sort_body,
        out_shape=[
            jax.ShapeDtypeStruct((B, 8, 128), jnp.float32),
            jax.ShapeDtypeStruct((B, 8, 128), jnp.int32),
        ],
        interpret=_INTERPRET,
    )(cv.reshape(B, 16, 128), ci.reshape(B, 16, 128), my, denom)


def kernel(saliency_map, K_arg, mask_logits):
    B, H, W = saliency_map.shape
    soft, gmax, my, denom = _stage_a(saliency_map, mask_logits)
    t = _stage_b(gmax)
    cv, ci = _stage_c_sc(saliency_map, mask_logits, gmax, t)
    scores, indices = _stage_d(cv, ci, my, denom)
    return (scores.reshape(B, K), indices.reshape(B, K), soft)
